# Initial kernel scaffold; baseline (speedup 1.0000x reference)
import jax
import jax.numpy as jnp
from jax.experimental import pallas as pl


def kernel(idx, w0, w, V):
    raise NotImplementedError("write your pallas kernel here")


# speedup vs baseline: 1.3169x; 1.3169x over previous
import functools

import jax
import jax.numpy as jnp
from jax import lax
from jax.experimental import pallas as pl
from jax.experimental.pallas import tpu as pltpu
from jax.experimental.pallas import tpu_sc as plsc

NC = 2
NS = 16
NW = NC * NS
LANES = 16


@functools.lru_cache(maxsize=None)
def _build(B, F, N, K, interpret=False):
    assert K == LANES
    assert B % NW == 0
    S = B // NW
    C = 64 if S % 64 == 0 else S
    NCH = S // C
    RPC = C * F
    GSZ = 128
    while RPC % GSZ:
        GSZ //= 2
    NSTR = RPC // GSZ

    mesh = plsc.VectorSubcoreMesh(
        core_axis_name="c", subcore_axis_name="s",
        num_cores=NC, num_subcores=NS)

    @functools.partial(
        pl.kernel,
        out_type=jax.ShapeDtypeStruct((B,), jnp.float32),
        mesh=mesh,
        scratch_types=[
            pltpu.VMEM((S * F,), jnp.int32),
            pltpu.VMEM((RPC, K), jnp.float32),
            pltpu.VMEM((RPC,), jnp.float32),
            pltpu.VMEM((S,), jnp.float32),
            pltpu.SemaphoreType.DMA,
            pltpu.SemaphoreType.DMA,
        ],
        compiler_params=pltpu.CompilerParams(
            needs_layout_passes=False, use_tc_tiling_on_sc=False),
        interpret=interpret,
    )
    def fm(idx_hbm, w_hbm, v_hbm, out_hbm, idx_v, rows_v, wv_v, out_v,
           sem_v, sem_w):
        wid = lax.axis_index("s") * NC + lax.axis_index("c")
        base = wid * (S * F)
        pltpu.sync_copy(idx_hbm.at[pl.ds(base, S * F)], idx_v)

        lane = lax.iota(jnp.int32, LANES)
        lane_f = lane * F
        last = lane == (LANES - 1)

        for g in range(NCH):
            cps = []
            for j in range(NSTR):
                isl = idx_v.at[pl.ds(g * RPC + j * GSZ, GSZ)]
                cps.append(pltpu.async_copy(
                    v_hbm.at[isl], rows_v.at[pl.ds(j * GSZ, GSZ)], sem_v))
                cps.append(pltpu.async_copy(
                    w_hbm.at[isl], wv_v.at[pl.ds(j * GSZ, GSZ)], sem_w))
            for cp in cps:
                cp.wait()

            def lin_body(gg, _):
                sbase = lane_f + gg * (LANES * F)
                lin = plsc.load_gather(wv_v, [sbase])
                for f in range(1, F):
                    lin = lin + plsc.load_gather(wv_v, [sbase + f])
                out_v[pl.ds(g * C + gg * LANES, LANES)] = lin
                return 0

            lax.fori_loop(0, C // LANES, lin_body, 0, unroll=False)

            def pair_body(s, _):
                rb = s * F
                r = rows_v[rb, :]
                acc = r
                acc2 = r * r
                for f in range(1, F):
                    r = rows_v[rb + f, :]
                    acc = acc + r
                    acc2 = acc2 + r * r
                t = acc * acc - acc2
                cum = plsc.cumsum(t) * 0.5
                pos = jnp.broadcast_to(g * C + s, (LANES,)).astype(jnp.int32)
                plsc.addupdate_scatter(out_v, [pos], cum, mask=last)
                return 0

            lax.fori_loop(0, C, pair_body, 0, unroll=False)

        pltpu.sync_copy(out_v, out_hbm.at[pl.ds(wid * S, S)])

    return fm


def kernel(idx, w0, w, V):
    B, F = idx.shape
    N, K = V.shape
    fm = _build(B, F, N, K)
    out = fm(idx.reshape(-1), w.reshape(-1), V)
    return out + w0[0]
